# Initial kernel scaffold; baseline (speedup 1.0000x reference)
import jax
import jax.numpy as jnp
from jax.experimental import pallas as pl


def kernel(x, table, W, b):
    raise NotImplementedError("write your pallas kernel here")


# speedup vs baseline: 2.3526x; 2.3526x over previous
import functools

import jax
import jax.numpy as jnp
from jax import lax
from jax.experimental import pallas as pl
from jax.experimental.pallas import tpu as pltpu
from jax.experimental.pallas import tpu_sc as plsc

_NC = 2
_NS = 16
_NW = _NC * _NS
_L = 16

_CHUNK = 128


def _matvec_body(tbl_ref, w_ref, b_ref, out_ref):
    out_ref[...] = jnp.sum(tbl_ref[...] * w_ref[...], axis=1) + b_ref[0]


def _project_table(table, w_row, b_scaled, blk):
    vocab, embed = table.shape
    grid = (vocab + blk - 1) // blk
    return pl.pallas_call(
        _matvec_body,
        grid=(grid,),
        in_specs=[
            pl.BlockSpec((blk, embed), lambda i: (i, 0)),
            pl.BlockSpec((1, embed), lambda i: (0, 0)),
            pl.BlockSpec(memory_space=pltpu.SMEM),
        ],
        out_specs=pl.BlockSpec((blk,), lambda i: (i,)),
        out_shape=jax.ShapeDtypeStruct((vocab,), jnp.float32),
    )(table, w_row, b_scaled)


def _make_sc_gather(batch, hist, ipw, nch):
    bpw = batch // _NW
    grp = bpw // _L
    mesh = plsc.VectorSubcoreMesh(
        core_axis_name="c", subcore_axis_name="s",
        num_cores=_NC, num_subcores=_NS,
    )

    @functools.partial(
        pl.kernel,
        mesh=mesh,
        out_type=jax.ShapeDtypeStruct((batch,), jnp.float32),
        scratch_types=[
            pltpu.VMEM((nch, _CHUNK), jnp.int32),
            pltpu.VMEM((ipw,), jnp.float32),
            pltpu.VMEM((bpw,), jnp.float32),
            pltpu.SemaphoreType.DMA,
        ],
    )
    def sc_gather(p_hbm, x_hbm, out_hbm, idx_v, vals_v, res_v, sem):
        wid = lax.axis_index("s") * _NC + lax.axis_index("c")
        pltpu.sync_copy(x_hbm.at[wid], idx_v)

        k = 8
        def fire_drain(g, carry):
            copies = []
            for i in range(k):
                j = g * k + i
                copies.append(
                    pltpu.async_copy(
                        p_hbm.at[idx_v.at[j]],
                        vals_v.at[pl.ds(j * _CHUNK, _CHUNK)],
                        sem,
                    )
                )
            for c in copies:
                c.wait()
            return carry

        lax.fori_loop(0, nch // k, fire_drain, 0)

        def grp_body(g, carry):
            acc = jnp.zeros((_L,), jnp.float32)
            for j in range(hist):
                acc = acc + vals_v[pl.ds(j * bpw + g * _L, _L)]
            res_v[pl.ds(g * _L, _L)] = acc
            return carry

        lax.fori_loop(0, grp, grp_body, 0)
        pltpu.sync_copy(res_v, out_hbm.at[pl.ds(wid * bpw, bpw)])

    return sc_gather


def kernel(x, table, W, b):
    batch, hist = x.shape
    vocab, embed = table.shape

    inv_h = 1.0 / hist
    w_row = (W * inv_h).reshape(1, embed)
    b_scaled = (b * inv_h).astype(jnp.float32)

    p = _project_table(table, w_row, b_scaled, blk=32768)

    bpw = batch // _NW
    ipw = bpw * hist
    nch = ipw // _CHUNK
    x3 = (x.astype(jnp.int32)
          .reshape(_NW, bpw, hist)
          .transpose(0, 2, 1)
          .reshape(_NW, nch, _CHUNK))

    out = _make_sc_gather(batch, hist, ipw, nch)(p, x3)
    return out.reshape(batch, 1)


# speedup vs baseline: 3.3194x; 1.4109x over previous
import functools

import jax
import jax.numpy as jnp
from jax import lax
from jax.experimental import pallas as pl
from jax.experimental.pallas import tpu as pltpu
from jax.experimental.pallas import tpu_sc as plsc

_NC = 2
_NS = 16
_NW = _NC * _NS
_L = 16

_CHUNK = 128


def _matvec_body(tbl_ref, w_ref, b_ref, out_ref):
    prod = jax.lax.dot_general(
        w_ref[...], tbl_ref[...],
        (((1,), (1,)), ((), ())),
        preferred_element_type=jnp.float32,
    )
    out_ref[...] = prod + b_ref[0]


def _project_table(table, w_row, b_scaled, blk):
    vocab, embed = table.shape
    grid = (vocab + blk - 1) // blk
    return pl.pallas_call(
        _matvec_body,
        grid=(grid,),
        in_specs=[
            pl.BlockSpec((blk, embed), lambda i: (i, 0)),
            pl.BlockSpec((1, embed), lambda i: (0, 0)),
            pl.BlockSpec(memory_space=pltpu.SMEM),
        ],
        out_specs=pl.BlockSpec((1, blk), lambda i: (0, i)),
        out_shape=jax.ShapeDtypeStruct((1, vocab), jnp.float32),
    )(table, w_row, b_scaled)


def _make_sc_gather(batch, hist, ipw, nch):
    bpw = batch // _NW
    grp = bpw // _L
    mesh = plsc.VectorSubcoreMesh(
        core_axis_name="c", subcore_axis_name="s",
        num_cores=_NC, num_subcores=_NS,
    )

    @functools.partial(
        pl.kernel,
        mesh=mesh,
        out_type=jax.ShapeDtypeStruct((batch,), jnp.float32),
        scratch_types=[
            pltpu.VMEM((nch, _CHUNK), jnp.int32),
            pltpu.VMEM((ipw,), jnp.float32),
            pltpu.VMEM((bpw,), jnp.float32),
            pltpu.SemaphoreType.DMA,
        ],
    )
    def sc_gather(p_hbm, x_hbm, out_hbm, idx_v, vals_v, res_v, sem):
        wid = lax.axis_index("s") * _NC + lax.axis_index("c")
        pltpu.sync_copy(x_hbm.at[wid], idx_v)

        k = 8
        def fire_drain(g, carry):
            copies = []
            for i in range(k):
                j = g * k + i
                copies.append(
                    pltpu.async_copy(
                        p_hbm.at[idx_v.at[j]],
                        vals_v.at[pl.ds(j * _CHUNK, _CHUNK)],
                        sem,
                    )
                )
            for c in copies:
                c.wait()
            return carry

        lax.fori_loop(0, nch // k, fire_drain, 0)

        def grp_body(g, carry):
            acc = jnp.zeros((_L,), jnp.float32)
            for j in range(hist):
                acc = acc + vals_v[pl.ds(j * bpw + g * _L, _L)]
            res_v[pl.ds(g * _L, _L)] = acc
            return carry

        lax.fori_loop(0, grp, grp_body, 0)
        pltpu.sync_copy(res_v, out_hbm.at[pl.ds(wid * bpw, bpw)])

    return sc_gather


def kernel(x, table, W, b):
    batch, hist = x.shape
    vocab, embed = table.shape

    inv_h = 1.0 / hist
    w_row = (W * inv_h).reshape(1, embed)
    b_scaled = (b * inv_h).astype(jnp.float32)

    p = _project_table(table, w_row, b_scaled, blk=32768).reshape(-1)

    bpw = batch // _NW
    ipw = bpw * hist
    nch = ipw // _CHUNK
    x3 = (x.astype(jnp.int32)
          .reshape(_NW, bpw, hist)
          .transpose(0, 2, 1)
          .reshape(_NW, nch, _CHUNK))

    out = _make_sc_gather(batch, hist, ipw, nch)(p, x3)
    return out.reshape(batch, 1)


# speedup vs baseline: 3.4638x; 1.0435x over previous
import functools

import jax
import jax.numpy as jnp
from jax import lax
from jax.experimental import pallas as pl
from jax.experimental.pallas import tpu as pltpu
from jax.experimental.pallas import tpu_sc as plsc

_NC = 2
_NS = 16
_NW = _NC * _NS
_L = 16

_CHUNK = 128


def _matvec_body(tbl_ref, w_ref, b_ref, out_ref):
    prod = jax.lax.dot_general(
        w_ref[...], tbl_ref[...],
        (((1,), (1,)), ((), ())),
        preferred_element_type=jnp.float32,
    )
    out_ref[...] = prod.reshape(-1) + b_ref[0]


def _project_table(table, w_row, b_scaled, blk):
    vocab, embed = table.shape
    grid = (vocab + blk - 1) // blk
    return pl.pallas_call(
        _matvec_body,
        grid=(grid,),
        in_specs=[
            pl.BlockSpec((blk, embed), lambda i: (i, 0)),
            pl.BlockSpec((1, embed), lambda i: (0, 0)),
            pl.BlockSpec(memory_space=pltpu.SMEM),
        ],
        out_specs=pl.BlockSpec((blk,), lambda i: (i,)),
        out_shape=jax.ShapeDtypeStruct((vocab,), jnp.float32),
    )(table, w_row, b_scaled)


def _transpose_body(x_ref, out_ref):
    out_ref[...] = jnp.swapaxes(x_ref[...], 1, 2)


def _transpose_indices(x4):
    nw, bpw, hist = x4.shape
    return pl.pallas_call(
        _transpose_body,
        grid=(nw,),
        in_specs=[pl.BlockSpec((1, bpw, hist), lambda i: (i, 0, 0))],
        out_specs=pl.BlockSpec((1, hist, bpw), lambda i: (i, 0, 0)),
        out_shape=jax.ShapeDtypeStruct((nw, hist, bpw), jnp.int32),
    )(x4)


def _make_sc_gather(batch, hist, ipw, nch):
    bpw = batch // _NW
    mesh = plsc.VectorSubcoreMesh(
        core_axis_name="c", subcore_axis_name="s",
        num_cores=_NC, num_subcores=_NS,
    )

    cpr = bpw // _CHUNK

    @functools.partial(
        pl.kernel,
        mesh=mesh,
        out_type=jax.ShapeDtypeStruct((batch,), jnp.float32),
        scratch_types=[
            pltpu.VMEM((hist, bpw), jnp.int32),
            pltpu.VMEM((ipw,), jnp.float32),
            pltpu.VMEM((bpw,), jnp.float32),
            pltpu.SemaphoreType.DMA,
        ],
    )
    def sc_gather(p_hbm, x_hbm, out_hbm, idx_v, vals_v, res_v, sem):
        wid = lax.axis_index("s") * _NC + lax.axis_index("c")
        pltpu.sync_copy(x_hbm.at[wid], idx_v)

        k = 8
        def fire_drain(g, carry):
            copies = []
            for i in range(k):
                c = g * k + i
                j, q = c // cpr, c % cpr
                copies.append(
                    pltpu.async_copy(
                        p_hbm.at[idx_v.at[j, pl.ds(q * _CHUNK, _CHUNK)]],
                        vals_v.at[pl.ds(c * _CHUNK, _CHUNK)],
                        sem,
                    )
                )
            for c in copies:
                c.wait()
            return carry

        lax.fori_loop(0, nch // k, fire_drain, 0)

        def grp_body(g, carry):
            acc = jnp.zeros((_L,), jnp.float32)
            for j in range(hist):
                acc = acc + vals_v[pl.ds(j * bpw + g * _L, _L)]
            res_v[pl.ds(g * _L, _L)] = acc
            return carry

        lax.fori_loop(0, bpw // _L, grp_body, 0)
        pltpu.sync_copy(res_v, out_hbm.at[pl.ds(wid * bpw, bpw)])

    return sc_gather


def kernel(x, table, W, b):
    batch, hist = x.shape
    vocab, embed = table.shape

    inv_h = 1.0 / hist
    w_row = (W * inv_h).reshape(1, embed)
    b_scaled = (b * inv_h).astype(jnp.float32)

    p = _project_table(table, w_row, b_scaled, blk=32768)

    bpw = batch // _NW
    ipw = bpw * hist
    nch = ipw // _CHUNK
    x4 = x.astype(jnp.int32).reshape(_NW, bpw, hist)
    xt = _transpose_indices(x4)

    out = _make_sc_gather(batch, hist, ipw, nch)(p, xt)
    return out.reshape(batch, 1)


# speedup vs baseline: 3.4667x; 1.0008x over previous
import functools

import jax
import jax.numpy as jnp
from jax import lax
from jax.experimental import pallas as pl
from jax.experimental.pallas import tpu as pltpu
from jax.experimental.pallas import tpu_sc as plsc

_NC = 2
_NS = 16
_NW = _NC * _NS
_L = 16

_CHUNK = 128


def _matvec_body(tbl_ref, w_ref, b_ref, out_ref):
    prod = jax.lax.dot_general(
        w_ref[...], tbl_ref[...],
        (((1,), (1,)), ((), ())),
        preferred_element_type=jnp.float32,
    )
    out_ref[...] = prod.reshape(-1) + b_ref[0]


def _project_table(table, w_row, b_scaled, blk):
    vocab, embed = table.shape
    grid = (vocab + blk - 1) // blk
    return pl.pallas_call(
        _matvec_body,
        grid=(grid,),
        in_specs=[
            pl.BlockSpec((blk, embed), lambda i: (i, 0)),
            pl.BlockSpec((1, embed), lambda i: (0, 0)),
            pl.BlockSpec(memory_space=pltpu.SMEM),
        ],
        out_specs=pl.BlockSpec((blk,), lambda i: (i,)),
        out_shape=jax.ShapeDtypeStruct((vocab,), jnp.float32),
    )(table, w_row, b_scaled)


def _transpose_body(x_ref, out_ref):
    out_ref[...] = jnp.swapaxes(x_ref[...], 1, 2).reshape(-1)


def _transpose_indices(x4):
    nw, bpw, hist = x4.shape
    return pl.pallas_call(
        _transpose_body,
        grid=(nw,),
        in_specs=[pl.BlockSpec((1, bpw, hist), lambda i: (i, 0, 0))],
        out_specs=pl.BlockSpec((hist * bpw,), lambda i: (i,)),
        out_shape=jax.ShapeDtypeStruct((nw * hist * bpw,), jnp.int32),
    )(x4)


def _make_sc_gather(batch, hist, ipw, nch):
    bpw = batch // _NW
    mesh = plsc.VectorSubcoreMesh(
        core_axis_name="c", subcore_axis_name="s",
        num_cores=_NC, num_subcores=_NS,
    )

    @functools.partial(
        pl.kernel,
        mesh=mesh,
        out_type=jax.ShapeDtypeStruct((batch,), jnp.float32),
        scratch_types=[
            pltpu.VMEM((ipw,), jnp.int32),
            pltpu.VMEM((ipw,), jnp.float32),
            pltpu.VMEM((bpw,), jnp.float32),
            pltpu.SemaphoreType.DMA,
        ],
    )
    def sc_gather(p_hbm, x_hbm, out_hbm, idx_v, vals_v, res_v, sem):
        wid = lax.axis_index("s") * _NC + lax.axis_index("c")
        pltpu.sync_copy(x_hbm.at[pl.ds(wid * ipw, ipw)], idx_v)

        k = 8
        def fire_drain(g, carry):
            copies = []
            for i in range(k):
                c = g * k + i
                copies.append(
                    pltpu.async_copy(
                        p_hbm.at[idx_v.at[pl.ds(c * _CHUNK, _CHUNK)]],
                        vals_v.at[pl.ds(c * _CHUNK, _CHUNK)],
                        sem,
                    )
                )
            for c in copies:
                c.wait()
            return carry

        lax.fori_loop(0, nch // k, fire_drain, 0)

        def grp_body(g, carry):
            acc = jnp.zeros((_L,), jnp.float32)
            for j in range(hist):
                acc = acc + vals_v[pl.ds(j * bpw + g * _L, _L)]
            res_v[pl.ds(g * _L, _L)] = acc
            return carry

        lax.fori_loop(0, bpw // _L, grp_body, 0)
        pltpu.sync_copy(res_v, out_hbm.at[pl.ds(wid * bpw, bpw)])

    return sc_gather


def kernel(x, table, W, b):
    batch, hist = x.shape
    vocab, embed = table.shape

    inv_h = 1.0 / hist
    w_row = (W * inv_h).reshape(1, embed)
    b_scaled = (b * inv_h).astype(jnp.float32)

    p = _project_table(table, w_row, b_scaled, blk=32768)

    bpw = batch // _NW
    ipw = bpw * hist
    nch = ipw // _CHUNK
    x4 = x.astype(jnp.int32).reshape(_NW, bpw, hist)
    xt = _transpose_indices(x4)

    out = _make_sc_gather(batch, hist, ipw, nch)(p, xt)
    return out.reshape(batch, 1)
